# baseline (speedup 1.0000x reference)
import functools

import jax
import jax.numpy as jnp
from jax import lax
from jax.experimental import pallas as pl
from jax.experimental.pallas import tpu as pltpu
from jax.experimental.pallas import tpu_sc as plsc

_INFO = plsc.get_sparse_core_info()
_NC, _NS, _L = _INFO.num_cores, _INFO.num_subcores, _INFO.num_lanes
_NW = _NC * _NS

_BB = 128


@functools.lru_cache(maxsize=None)
def _build(b, s_len, d):
    nblk = b // _BB
    blk_per_w = nblk // _NW
    d_hi, d_lo = d // 8, 8
    assert s_len % 2 == 0

    mesh = plsc.VectorSubcoreMesh(core_axis_name="c", subcore_axis_name="s")

    @functools.partial(
        pl.kernel,
        out_type=jax.ShapeDtypeStruct((s_len, d_hi, nblk, d_lo, _BB), jnp.float32),
        mesh=mesh,
        scratch_types=[
            pltpu.VMEM((_BB, s_len), jnp.int32),
            pltpu.VMEM((s_len, _BB), jnp.int32),
            pltpu.VMEM((2, _BB, d), jnp.float32),
            pltpu.VMEM((2, d_hi, d_lo, _BB), jnp.float32),
            pltpu.SemaphoreType.DMA,
            pltpu.SemaphoreType.DMA,
            pltpu.SemaphoreType.DMA,
            pltpu.SemaphoreType.DMA,
        ],
        compiler_params=pltpu.CompilerParams(
            use_tc_tiling_on_sc=False, needs_layout_passes=False
        ),
    )
    def k(tid_hbm, table_hbm, o5_hbm, tid_raw, idx_t, rows_v, obuf, g0, g1, w0, w1):
        gsem = (g0, g1)
        wsem = (w0, w1)
        wid = lax.axis_index("c") * _NS + lax.axis_index("s")
        iota = lax.iota(jnp.int32, _L)
        row_idx = [iota + b16 * _L for b16 in range(_BB // _L)]

        def fire_gather(s, buf, jb):
            pltpu.async_copy(
                table_hbm.at[idx_t.at[s]], rows_v.at[buf], gsem[buf]
            )

        def wait_gather(buf):
            pltpu.make_async_copy(
                table_hbm.at[pl.ds(0, _BB)], rows_v.at[buf], gsem[buf]
            ).wait()

        def fire_writeback(s, buf, jb):
            pltpu.async_copy(
                obuf.at[buf], o5_hbm.at[s, :, jb], wsem[buf]
            )

        def wait_writeback(buf):
            pltpu.make_async_copy(
                obuf.at[buf], o5_hbm.at[0, :, 0], wsem[buf]
            ).wait()

        def transpose_unit(buf):
            def ibody(i, carry):
                for dlo in range(d_lo):
                    col = jnp.full((_L,), 0, jnp.int32) + (i * 8 + dlo)
                    for b16 in range(_BB // _L):
                        v = plsc.load_gather(
                            rows_v.at[buf], [row_idx[b16], col]
                        )
                        obuf[buf, i, dlo, pl.ds(b16 * _L, _L)] = v
                return carry

            lax.fori_loop(0, d_hi, ibody, 0)

        for jbi in range(blk_per_w):
            jb = wid * blk_per_w + jbi

            pltpu.sync_copy(tid_hbm.at[pl.ds(jb * _BB, _BB)], tid_raw)

            def sbody(s, carry):
                scol = jnp.full((_L,), 0, jnp.int32) + s
                for b16 in range(_BB // _L):
                    v = plsc.load_gather(tid_raw, [row_idx[b16], scol])
                    idx_t[s, pl.ds(b16 * _L, _L)] = v
                return carry

            lax.fori_loop(0, s_len, sbody, 0)

            fire_gather(0, 0, jb)

            def body(ii, carry):
                for buf in range(2):
                    s = ii * 2 + buf
                    nb = 1 - buf

                    def _wait_prev_wb():
                        wait_writeback(nb)

                    if jbi == 0 and buf == 0:
                        pl.when(s >= 2)(_wait_prev_wb)
                    else:
                        _wait_prev_wb()

                    def _fire_next():
                        fire_gather(s + 1, nb, jb)

                    pl.when(s + 1 < s_len)(_fire_next)
                    wait_gather(buf)
                    transpose_unit(buf)
                    fire_writeback(s, buf, jb)
                return carry

            lax.fori_loop(0, s_len // 2, body, 0)
        wait_writeback(1)

    return k


def kernel(token_ids, weight):
    b, s_len = token_ids.shape
    d = weight.shape[1]
    o5 = _build(b, s_len, d)(token_ids.astype(jnp.int32), weight)
    o = jnp.transpose(o5, (2, 4, 0, 1, 3))
    return o.reshape(b, s_len, d)


# speedup vs baseline: 1.0058x; 1.0058x over previous
import functools

import jax
import jax.numpy as jnp
from jax import lax
from jax.experimental import pallas as pl
from jax.experimental.pallas import tpu as pltpu
from jax.experimental.pallas import tpu_sc as plsc

_INFO = plsc.get_sparse_core_info()
_NC, _NS, _L = _INFO.num_cores, _INFO.num_subcores, _INFO.num_lanes
_NW = _NC * _NS

_BB = 128


@functools.lru_cache(maxsize=None)
def _build(b, s_len, d):
    nblk = b // _BB
    blk_per_w = nblk // _NW
    d_hi, d_lo = d // 8, 8
    assert s_len % 2 == 0

    mesh = plsc.VectorSubcoreMesh(core_axis_name="c", subcore_axis_name="s")

    @functools.partial(
        pl.kernel,
        out_type=jax.ShapeDtypeStruct((s_len, d_hi, nblk, d_lo, _BB), jnp.float32),
        mesh=mesh,
        scratch_types=[
            pltpu.VMEM((_BB, s_len), jnp.int32),
            pltpu.VMEM((s_len, _BB), jnp.int32),
            pltpu.VMEM((2, _BB, d), jnp.float32),
            pltpu.VMEM((2, d_hi, d_lo, _BB), jnp.float32),
            pltpu.SemaphoreType.DMA,
            pltpu.SemaphoreType.DMA,
            pltpu.SemaphoreType.DMA,
            pltpu.SemaphoreType.DMA,
        ],
        compiler_params=pltpu.CompilerParams(
            use_tc_tiling_on_sc=False, needs_layout_passes=False
        ),
    )
    def k(tid_hbm, table_hbm, o5_hbm, tid_raw, idx_t, rows_v, obuf, g0, g1, w0, w1):
        gsem = (g0, g1)
        wsem = (w0, w1)
        wid = lax.axis_index("c") * _NS + lax.axis_index("s")
        iota = lax.iota(jnp.int32, _L)
        row_idx = [iota + b16 * _L for b16 in range(_BB // _L)]

        def fire_gather(s, buf, jb):
            pltpu.async_copy(
                table_hbm.at[idx_t.at[s]], rows_v.at[buf], gsem[buf]
            )

        def wait_gather(buf):
            pltpu.make_async_copy(
                table_hbm.at[pl.ds(0, _BB)], rows_v.at[buf], gsem[buf]
            ).wait()

        def fire_writeback(s, buf, jb):
            pltpu.async_copy(
                obuf.at[buf], o5_hbm.at[s, :, jb], wsem[buf]
            )

        def wait_writeback(buf):
            pltpu.make_async_copy(
                obuf.at[buf], o5_hbm.at[0, :, 0], wsem[buf]
            ).wait()

        def transpose_unit(buf):
            def ibody(i, carry):
                base = i * d_lo
                for dlo in range(d_lo):
                    col = jnp.full((_L,), 0, jnp.int32) + (base + dlo)
                    for b16 in range(_BB // _L):
                        v = plsc.load_gather(
                            rows_v.at[buf], [row_idx[b16], col]
                        )
                        obuf[buf, i, dlo, pl.ds(b16 * _L, _L)] = v
                return carry

            lax.fori_loop(0, d_hi, ibody, 0, unroll=2)

        for jbi in range(blk_per_w):
            jb = wid * blk_per_w + jbi

            pltpu.sync_copy(tid_hbm.at[pl.ds(jb * _BB, _BB)], tid_raw)

            def sbody(s, carry):
                scol = jnp.full((_L,), 0, jnp.int32) + s
                for b16 in range(_BB // _L):
                    v = plsc.load_gather(tid_raw, [row_idx[b16], scol])
                    idx_t[s, pl.ds(b16 * _L, _L)] = v
                return carry

            lax.fori_loop(0, s_len, sbody, 0)

            fire_gather(0, 0, jb)

            def body(ii, carry):
                for buf in range(2):
                    s = ii * 2 + buf
                    nb = 1 - buf

                    def _wait_prev_wb():
                        wait_writeback(nb)

                    if jbi == 0 and buf == 0:
                        pl.when(s >= 2)(_wait_prev_wb)
                    else:
                        _wait_prev_wb()

                    def _fire_next():
                        fire_gather(s + 1, nb, jb)

                    pl.when(s + 1 < s_len)(_fire_next)
                    wait_gather(buf)
                    transpose_unit(buf)
                    fire_writeback(s, buf, jb)
                return carry

            lax.fori_loop(0, s_len // 2, body, 0)
        wait_writeback(1)

    return k


def kernel(token_ids, weight):
    b, s_len = token_ids.shape
    d = weight.shape[1]
    o5 = _build(b, s_len, d)(token_ids.astype(jnp.int32), weight)
    o = jnp.transpose(o5, (2, 4, 0, 1, 3))
    return o.reshape(b, s_len, d)


# speedup vs baseline: 1.8124x; 1.8019x over previous
import functools

import jax
import jax.numpy as jnp
from jax import lax
from jax.experimental import pallas as pl
from jax.experimental.pallas import tpu as pltpu
from jax.experimental.pallas import tpu_sc as plsc

_INFO = plsc.get_sparse_core_info()
_NC, _NS, _L = _INFO.num_cores, _INFO.num_subcores, _INFO.num_lanes
_NW = _NC * _NS

_BB = 128


@functools.lru_cache(maxsize=None)
def _build(b, s_len, d):
    nblk = b // _BB
    blk_per_w = nblk // _NW
    d_hi, d_lo = d // 8, 8
    assert s_len % 2 == 0

    mesh = plsc.VectorSubcoreMesh(core_axis_name="c", subcore_axis_name="s")

    @functools.partial(
        pl.kernel,
        out_type=jax.ShapeDtypeStruct((s_len, d_hi, nblk, d_lo, _BB), jnp.float32),
        mesh=mesh,
        scratch_types=[
            pltpu.VMEM((_BB, s_len), jnp.int32),
            pltpu.VMEM((s_len, _BB), jnp.int32),
            pltpu.VMEM((2, _BB, d), jnp.float32),
            pltpu.VMEM((2, d_hi, d_lo, _BB + 1), jnp.float32),
            pltpu.SemaphoreType.DMA,
            pltpu.SemaphoreType.DMA,
            pltpu.SemaphoreType.DMA,
            pltpu.SemaphoreType.DMA,
        ],
        compiler_params=pltpu.CompilerParams(
            use_tc_tiling_on_sc=False, needs_layout_passes=False
        ),
    )
    def k(tid_hbm, table_hbm, o5_hbm, tid_raw, idx_t, rows_v, obuf, g0, g1, w0, w1):
        gsem = (g0, g1)
        wsem = (w0, w1)
        wid = lax.axis_index("c") * _NS + lax.axis_index("s")
        iota = lax.iota(jnp.int32, _L)
        row_idx = [iota + b16 * _L for b16 in range(_BB // _L)]

        def fire_gather(s, buf, jb):
            pltpu.async_copy(
                table_hbm.at[idx_t.at[s]], rows_v.at[buf], gsem[buf]
            )

        def wait_gather(buf):
            pltpu.make_async_copy(
                table_hbm.at[pl.ds(0, _BB)], rows_v.at[buf], gsem[buf]
            ).wait()

        def fire_writeback(s, buf, jb):
            for i in range(d_hi):
                pltpu.async_copy(
                    obuf.at[buf, i, :, pl.ds(0, _BB)],
                    o5_hbm.at[s, i, jb],
                    wsem[buf],
                )

        def wait_writeback(buf):
            pltpu.make_async_copy(
                obuf.at[buf, :, :, pl.ds(0, _BB)],
                o5_hbm.at[0, :, 0],
                wsem[buf],
            ).wait()

        ivecs = [(iota + 16 * kk) // d_lo for kk in range(d // _L)]
        dvecs = [(iota + 16 * kk) % d_lo for kk in range(d // _L)]

        def transpose_unit(buf):
            def gbody(g, carry):
                for j in range(_L):
                    bp = g * _L + j
                    bvec = jnp.full((_L,), 0, jnp.int32) + bp
                    for kk in range(d // _L):
                        v = rows_v[buf, bp, pl.ds(_L * kk, _L)]
                        plsc.store_scatter(
                            obuf.at[buf], [ivecs[kk], dvecs[kk], bvec], v
                        )
                return carry

            lax.fori_loop(0, _BB // _L, gbody, 0)

        for jbi in range(blk_per_w):
            jb = wid * blk_per_w + jbi

            pltpu.sync_copy(tid_hbm.at[pl.ds(jb * _BB, _BB)], tid_raw)

            def sbody(s, carry):
                scol = jnp.full((_L,), 0, jnp.int32) + s
                for b16 in range(_BB // _L):
                    v = plsc.load_gather(tid_raw, [row_idx[b16], scol])
                    idx_t[s, pl.ds(b16 * _L, _L)] = v
                return carry

            lax.fori_loop(0, s_len, sbody, 0)

            fire_gather(0, 0, jb)

            def body(ii, carry):
                for buf in range(2):
                    s = ii * 2 + buf
                    nb = 1 - buf

                    def _wait_prev_wb():
                        wait_writeback(nb)

                    if jbi == 0 and buf == 0:
                        pl.when(s >= 2)(_wait_prev_wb)
                    else:
                        _wait_prev_wb()

                    def _fire_next():
                        fire_gather(s + 1, nb, jb)

                    pl.when(s + 1 < s_len)(_fire_next)
                    wait_gather(buf)
                    transpose_unit(buf)
                    fire_writeback(s, buf, jb)
                return carry

            lax.fori_loop(0, s_len // 2, body, 0)
        wait_writeback(1)

    return k


def kernel(token_ids, weight):
    b, s_len = token_ids.shape
    d = weight.shape[1]
    o5 = _build(b, s_len, d)(token_ids.astype(jnp.int32), weight)
    o = jnp.transpose(o5, (2, 4, 0, 1, 3))
    return o.reshape(b, s_len, d)


# speedup vs baseline: 1.8216x; 1.0051x over previous
import functools

import jax
import jax.numpy as jnp
from jax import lax
from jax.experimental import pallas as pl
from jax.experimental.pallas import tpu as pltpu
from jax.experimental.pallas import tpu_sc as plsc

_INFO = plsc.get_sparse_core_info()
_NC, _NS, _L = _INFO.num_cores, _INFO.num_subcores, _INFO.num_lanes
_NW = _NC * _NS

_BB = 128


@functools.lru_cache(maxsize=None)
def _build(b, s_len, d):
    nblk = b // _BB
    blk_per_w = nblk // _NW
    d_hi, d_lo = d // 8, 8
    assert s_len % 2 == 0

    mesh = plsc.VectorSubcoreMesh(core_axis_name="c", subcore_axis_name="s")

    @functools.partial(
        pl.kernel,
        out_type=jax.ShapeDtypeStruct((s_len, d_hi, nblk, d_lo, _BB), jnp.float32),
        mesh=mesh,
        scratch_types=[
            pltpu.VMEM((_BB, s_len), jnp.int32),
            pltpu.VMEM((s_len, _BB), jnp.int32),
            pltpu.VMEM((2, _BB, d), jnp.float32),
            pltpu.VMEM((2, d_hi, d_lo, _BB + 1), jnp.float32),
            pltpu.SemaphoreType.DMA,
            pltpu.SemaphoreType.DMA,
            pltpu.SemaphoreType.DMA,
            pltpu.SemaphoreType.DMA,
        ],
        compiler_params=pltpu.CompilerParams(
            use_tc_tiling_on_sc=False, needs_layout_passes=False
        ),
    )
    def k(tid_hbm, table_hbm, o5_hbm, tid_raw, idx_t, rows_v, obuf, g0, g1, w0, w1):
        gsem = (g0, g1)
        wsem = (w0, w1)
        wid = lax.axis_index("c") * _NS + lax.axis_index("s")
        iota = lax.iota(jnp.int32, _L)
        row_idx = [iota + b16 * _L for b16 in range(_BB // _L)]

        def fire_gather(s, buf, jb):
            pltpu.async_copy(
                table_hbm.at[idx_t.at[s]], rows_v.at[buf], gsem[buf]
            )

        def wait_gather(buf):
            pltpu.make_async_copy(
                table_hbm.at[pl.ds(0, _BB)], rows_v.at[buf], gsem[buf]
            ).wait()

        def fire_writeback(s, buf, jb):
            for i in range(d_hi):
                pltpu.async_copy(
                    obuf.at[buf, i, :, pl.ds(0, _BB)],
                    o5_hbm.at[s, i, jb],
                    wsem[buf],
                )

        def wait_writeback(buf):
            pltpu.make_async_copy(
                obuf.at[buf, :, :, pl.ds(0, _BB)],
                o5_hbm.at[0, :, 0],
                wsem[buf],
            ).wait()

        ivecs = [(iota + 16 * kk) // d_lo for kk in range(d // _L)]
        dvecs = [(iota + 16 * kk) % d_lo for kk in range(d // _L)]

        def transpose_unit(buf):
            def gbody(g, carry):
                for j in range(_L):
                    bp = g * _L + j
                    bvec = jnp.full((_L,), 0, jnp.int32) + bp
                    for kk in range(d // _L):
                        v = rows_v[buf, bp, pl.ds(_L * kk, _L)]
                        plsc.store_scatter(
                            obuf.at[buf], [ivecs[kk], dvecs[kk], bvec], v
                        )
                return carry

            lax.fori_loop(0, _BB // _L, gbody, 0, unroll=2)

        for jbi in range(blk_per_w):
            jb = wid * blk_per_w + jbi

            pltpu.sync_copy(tid_hbm.at[pl.ds(jb * _BB, _BB)], tid_raw)

            def sbody(s, carry):
                scol = jnp.full((_L,), 0, jnp.int32) + s
                for b16 in range(_BB // _L):
                    v = plsc.load_gather(tid_raw, [row_idx[b16], scol])
                    idx_t[s, pl.ds(b16 * _L, _L)] = v
                return carry

            lax.fori_loop(0, s_len, sbody, 0)

            fire_gather(0, 0, jb)

            def body(ii, carry):
                for buf in range(2):
                    s = ii * 2 + buf
                    nb = 1 - buf

                    def _wait_prev_wb():
                        wait_writeback(nb)

                    if jbi == 0 and buf == 0:
                        pl.when(s >= 2)(_wait_prev_wb)
                    else:
                        _wait_prev_wb()

                    def _fire_next():
                        fire_gather(s + 1, nb, jb)

                    pl.when(s + 1 < s_len)(_fire_next)
                    wait_gather(buf)
                    transpose_unit(buf)
                    fire_writeback(s, buf, jb)
                return carry

            lax.fori_loop(0, s_len // 2, body, 0)
        wait_writeback(1)

    return k


def kernel(token_ids, weight):
    b, s_len = token_ids.shape
    d = weight.shape[1]
    o5 = _build(b, s_len, d)(token_ids.astype(jnp.int32), weight)
    o = jnp.transpose(o5, (2, 4, 0, 1, 3))
    return o.reshape(b, s_len, d)


# speedup vs baseline: 1.8250x; 1.0018x over previous
import functools

import jax
import jax.numpy as jnp
from jax import lax
from jax.experimental import pallas as pl
from jax.experimental.pallas import tpu as pltpu
from jax.experimental.pallas import tpu_sc as plsc

_INFO = plsc.get_sparse_core_info()
_NC, _NS, _L = _INFO.num_cores, _INFO.num_subcores, _INFO.num_lanes
_NW = _NC * _NS

_BB = 128


@functools.lru_cache(maxsize=None)
def _build(b, s_len, d):
    nblk = b // _BB
    blk_per_w = nblk // _NW
    d_hi, d_lo = d // 8, 8
    assert s_len % 2 == 0

    mesh = plsc.VectorSubcoreMesh(core_axis_name="c", subcore_axis_name="s")

    @functools.partial(
        pl.kernel,
        out_type=jax.ShapeDtypeStruct((s_len, d_hi, nblk, d_lo, _BB), jnp.float32),
        mesh=mesh,
        scratch_types=[
            pltpu.VMEM((_BB, s_len), jnp.int32),
            pltpu.VMEM((s_len, _BB), jnp.int32),
            pltpu.VMEM((2, _BB, d), jnp.float32),
            pltpu.VMEM((2, d_hi, d_lo, _BB + 1), jnp.float32),
            pltpu.SemaphoreType.DMA,
            pltpu.SemaphoreType.DMA,
            pltpu.SemaphoreType.DMA,
            pltpu.SemaphoreType.DMA,
        ],
        compiler_params=pltpu.CompilerParams(
            use_tc_tiling_on_sc=False, needs_layout_passes=False
        ),
    )
    def k(tid_hbm, table_hbm, o5_hbm, tid_raw, idx_t, rows_v, obuf, g0, g1, w0, w1):
        gsem = (g0, g1)
        wsem = (w0, w1)
        wid = lax.axis_index("c") * _NS + lax.axis_index("s")
        iota = lax.iota(jnp.int32, _L)
        row_idx = [iota + b16 * _L for b16 in range(_BB // _L)]

        def fire_gather(s, buf, jb):
            pltpu.async_copy(
                table_hbm.at[idx_t.at[s]], rows_v.at[buf], gsem[buf]
            )

        def wait_gather(buf):
            pltpu.make_async_copy(
                table_hbm.at[pl.ds(0, _BB)], rows_v.at[buf], gsem[buf]
            ).wait()

        def fire_writeback(s, buf, jb):
            pltpu.async_copy(
                obuf.at[buf, :, :, pl.ds(0, _BB)],
                o5_hbm.at[s, :, jb],
                wsem[buf],
            )

        def wait_writeback(buf):
            pltpu.make_async_copy(
                obuf.at[buf, :, :, pl.ds(0, _BB)],
                o5_hbm.at[0, :, 0],
                wsem[buf],
            ).wait()

        ivecs = [(iota + 16 * kk) // d_lo for kk in range(d // _L)]
        dvecs = [(iota + 16 * kk) % d_lo for kk in range(d // _L)]

        def transpose_unit(buf):
            def gbody(g, carry):
                for j in range(_L):
                    bp = g * _L + j
                    bvec = jnp.full((_L,), 0, jnp.int32) + bp
                    for kk in range(d // _L):
                        v = rows_v[buf, bp, pl.ds(_L * kk, _L)]
                        plsc.store_scatter(
                            obuf.at[buf], [ivecs[kk], dvecs[kk], bvec], v
                        )
                return carry

            lax.fori_loop(0, _BB // _L, gbody, 0, unroll=2)

        for jbi in range(blk_per_w):
            jb = wid * blk_per_w + jbi

            pltpu.sync_copy(tid_hbm.at[pl.ds(jb * _BB, _BB)], tid_raw)

            def sbody(s, carry):
                scol = jnp.full((_L,), 0, jnp.int32) + s
                for b16 in range(_BB // _L):
                    v = plsc.load_gather(tid_raw, [row_idx[b16], scol])
                    idx_t[s, pl.ds(b16 * _L, _L)] = v
                return carry

            lax.fori_loop(0, s_len, sbody, 0)

            fire_gather(0, 0, jb)

            def body(ii, carry):
                for buf in range(2):
                    s = ii * 2 + buf
                    nb = 1 - buf

                    def _wait_prev_wb():
                        wait_writeback(nb)

                    if jbi == 0 and buf == 0:
                        pl.when(s >= 2)(_wait_prev_wb)
                    else:
                        _wait_prev_wb()

                    def _fire_next():
                        fire_gather(s + 1, nb, jb)

                    pl.when(s + 1 < s_len)(_fire_next)
                    wait_gather(buf)
                    transpose_unit(buf)
                    fire_writeback(s, buf, jb)
                return carry

            lax.fori_loop(0, s_len // 2, body, 0)
        wait_writeback(1)

    return k


def kernel(token_ids, weight):
    b, s_len = token_ids.shape
    d = weight.shape[1]
    o5 = _build(b, s_len, d)(token_ids.astype(jnp.int32), weight)
    o = jnp.transpose(o5, (2, 4, 0, 1, 3))
    return o.reshape(b, s_len, d)
